# Initial kernel scaffold; baseline (speedup 1.0000x reference)
import jax
import jax.numpy as jnp
from jax.experimental import pallas as pl


def kernel(feature_s, xyz_s, feature_t, xyz_t, Ws, bs, gamma_s, beta_s, Wt, bt, gamma_t, beta_t):
    raise NotImplementedError("write your pallas kernel here")


# speedup vs baseline: 7.5971x; 7.5971x over previous
import functools

import jax
import jax.numpy as jnp
from jax import lax
from jax.experimental import pallas as pl
from jax.experimental.pallas import tpu as pltpu
from jax.experimental.pallas import tpu_sc as plsc

K_NN = 12
S_PTS = 32
_EPS = 1e-5

_SC_CORES = 2
_SC_SUBCORES = 16
_SC_WORKERS = _SC_CORES * _SC_SUBCORES


def _fps_knn_body(n, xyzt_ref, xyzs_ref, idxt_ref, idxs_ref):
    b = pl.program_id(0)
    xt = xyzt_ref[0, 0:1, :]
    yt = xyzt_ref[0, 1:2, :]
    zt = xyzt_ref[0, 2:3, :]
    xs = xyzs_ref[0, 0:1, :]
    ys = xyzs_ref[0, 1:2, :]
    zs = xyzs_ref[0, 2:3, :]

    iota = lax.broadcasted_iota(jnp.int32, (1, n), 1)
    dist = jnp.full((1, n), 1e10, dtype=jnp.float32)
    far = jnp.int32(0)
    centroids = []
    for _ in range(S_PTS):
        onehot = iota == far
        cx = jnp.sum(jnp.where(onehot, xt, 0.0))
        cy = jnp.sum(jnp.where(onehot, yt, 0.0))
        cz = jnp.sum(jnp.where(onehot, zt, 0.0))
        centroids.append((cx, cy, cz))
        dx = xt - cx
        dy = yt - cy
        dz = zt - cz
        d = dx * dx + dy * dy + dz * dz
        dist = jnp.minimum(dist, d)
        m = jnp.max(dist)
        far = jnp.min(jnp.where(dist == m, iota, n))

    rows = []
    for cx, cy, cz in centroids:
        dx = xt - cx
        dy = yt - cy
        dz = zt - cz
        rows.append(dx * dx + dy * dy + dz * dz)
    for cx, cy, cz in centroids:
        dx = xs - cx
        dy = ys - cy
        dz = zs - cz
        rows.append(dx * dx + dy * dy + dz * dz)
    dmat = jnp.concatenate(rows, axis=0)

    iota2 = lax.broadcasted_iota(jnp.int32, (2 * S_PTS, n), 1)
    cols = []
    for _ in range(K_NN):
        m = jnp.min(dmat, axis=1, keepdims=True)
        sel = jnp.min(jnp.where(dmat == m, iota2, n), axis=1, keepdims=True)
        cols.append(sel)
        dmat = jnp.where(iota2 == sel, jnp.float32(3.4e38), dmat)
    idx = jnp.concatenate(cols, axis=1)

    base = b * n
    idxt_ref[0] = idx[0:S_PTS, :] + base
    idxs_ref[0] = idx[S_PTS : 2 * S_PTS, :] + base


def _fps_knn(xyzt_T, xyzs_T):
    bsz, _, n = xyzt_T.shape
    out_shape = jax.ShapeDtypeStruct((bsz, S_PTS, K_NN), jnp.int32)
    return pl.pallas_call(
        functools.partial(_fps_knn_body, n),
        grid=(bsz,),
        in_specs=[
            pl.BlockSpec((1, 3, n), lambda b: (b, 0, 0)),
            pl.BlockSpec((1, 3, n), lambda b: (b, 0, 0)),
        ],
        out_specs=[
            pl.BlockSpec((1, S_PTS, K_NN), lambda b: (b, 0, 0)),
            pl.BlockSpec((1, S_PTS, K_NN), lambda b: (b, 0, 0)),
        ],
        out_shape=[out_shape, out_shape],
    )(xyzt_T, xyzs_T)


def _gather_rows(tbl_s, flat_s, tbl_t, flat_t):
    n_rows = flat_s.shape[0]
    per = n_rows // _SC_WORKERS
    cs = tbl_s.shape[1]
    ct = tbl_t.shape[1]
    mesh = plsc.VectorSubcoreMesh(
        core_axis_name="c",
        subcore_axis_name="s",
        num_cores=_SC_CORES,
        num_subcores=_SC_SUBCORES,
    )

    @functools.partial(
        pl.kernel,
        mesh=mesh,
        out_type=[
            jax.ShapeDtypeStruct((n_rows, cs), jnp.float32),
            jax.ShapeDtypeStruct((n_rows, ct), jnp.float32),
        ],
        scratch_types=[
            pltpu.VMEM((per,), jnp.int32),
            pltpu.VMEM((per, cs), jnp.float32),
            pltpu.VMEM((per,), jnp.int32),
            pltpu.VMEM((per, ct), jnp.float32),
            pltpu.SemaphoreType.DMA,
            pltpu.SemaphoreType.DMA,
        ],
    )
    def gather_k(tbls, idxs, tblt, idxt, out_s, out_t,
                 idxv_s, rows_s, idxv_t, rows_t, sem_s, sem_t):
        wid = lax.axis_index("s") * _SC_CORES + lax.axis_index("c")
        base = wid * per
        pltpu.sync_copy(idxs.at[pl.ds(base, per)], idxv_s)
        pltpu.sync_copy(idxt.at[pl.ds(base, per)], idxv_t)
        cp_s = pltpu.async_copy(tbls.at[idxv_s], rows_s, sem_s)
        cp_t = pltpu.async_copy(tblt.at[idxv_t], rows_t, sem_t)
        cp_s.wait()
        pltpu.sync_copy(rows_s, out_s.at[pl.ds(base, per)])
        cp_t.wait()
        pltpu.sync_copy(rows_t, out_t.at[pl.ds(base, per)])

    return gather_k(tbl_s, flat_s, tbl_t, flat_t)


def _dense_body(gs_ref, gt_ref, ws_ref, wt_ref, ps_ref, pt_ref,
                outs_ref, outt_ref):
    def branch(g, w, p, out_ref):
        y = lax.dot_general(g, w, (((1,), (1,)), ((), ())),
                            preferred_element_type=jnp.float32)
        y = y + p[0:1, :]
        mean = jnp.mean(y, axis=0, keepdims=True)
        c = y - mean
        var = jnp.mean(c * c, axis=0, keepdims=True)
        z = p[1:2, :] * (c / jnp.sqrt(var + _EPS)) + p[2:3, :]
        z = jnp.maximum(z, 0.0)
        nrow = out_ref.shape[0]
        acc = z[0:nrow, :]
        for k in range(1, K_NN):
            acc = jnp.maximum(acc, z[k * nrow : (k + 1) * nrow, :])
        out_ref[...] = acc

    branch(gs_ref[...], ws_ref[...], ps_ref[...], outs_ref)
    branch(gt_ref[...], wt_ref[...], pt_ref[...], outt_ref)


def _dense(g_s, g_t, Ws, Wt, ps, pt, n_groups):
    n_rows, cs = g_s.shape
    ct = g_t.shape[1]
    o = Ws.shape[0]
    ch = 256
    grid = o // ch
    return pl.pallas_call(
        _dense_body,
        grid=(grid,),
        in_specs=[
            pl.BlockSpec((n_rows, cs), lambda j: (0, 0)),
            pl.BlockSpec((n_rows, ct), lambda j: (0, 0)),
            pl.BlockSpec((ch, cs), lambda j: (j, 0)),
            pl.BlockSpec((ch, ct), lambda j: (j, 0)),
            pl.BlockSpec((3, ch), lambda j: (0, j)),
            pl.BlockSpec((3, ch), lambda j: (0, j)),
        ],
        out_specs=[
            pl.BlockSpec((n_groups, ch), lambda j: (0, j)),
            pl.BlockSpec((n_groups, ch), lambda j: (0, j)),
        ],
        out_shape=[
            jax.ShapeDtypeStruct((n_groups, o), jnp.float32),
            jax.ShapeDtypeStruct((n_groups, o), jnp.float32),
        ],
    )(g_s, g_t, Ws, Wt, ps, pt)


def kernel(feature_s, xyz_s, feature_t, xyz_t,
           Ws, bs, gamma_s, beta_s, Wt, bt, gamma_t, beta_t):
    bsz, n, cs = feature_s.shape
    ct = feature_t.shape[2]
    o = Ws.shape[0]

    xyzt_T = jnp.transpose(xyz_t, (0, 2, 1))
    xyzs_T = jnp.transpose(xyz_s, (0, 2, 1))
    idx_t, idx_s = _fps_knn(xyzt_T, xyzs_T)

    flat_t = jnp.transpose(idx_t, (2, 0, 1)).reshape(-1)
    flat_s = jnp.transpose(idx_s, (2, 0, 1)).reshape(-1)

    g_s, g_t = _gather_rows(
        feature_s.reshape(bsz * n, cs), flat_s,
        feature_t.reshape(bsz * n, ct), flat_t,
    )

    ps = jnp.stack([bs, gamma_s, beta_s])
    pt = jnp.stack([bt, gamma_t, beta_t])
    n_groups = bsz * S_PTS
    out_s, out_t = _dense(g_s, g_t, Ws, Wt, ps, pt, n_groups)
    return (out_s.reshape(bsz, S_PTS, o), out_t.reshape(bsz, S_PTS, o))
